# Initial kernel scaffold; baseline (speedup 1.0000x reference)
import jax
import jax.numpy as jnp
from jax.experimental import pallas as pl


def kernel(x, edge_index, W1, b1, W2, b2):
    raise NotImplementedError("write your pallas kernel here")


# speedup vs baseline: 10.8140x; 10.8140x over previous
import functools

import jax
import jax.numpy as jnp
from jax import lax
from jax.experimental import pallas as pl
from jax.experimental.pallas import tpu as pltpu
from jax.experimental.pallas import tpu_sc as plsc

_N = 10000
_E = 160000
_DIN = 256
_DH = 256
_DOUT = 64
_W = 128
_BN = 1000
_CH = 128

_ZR = 208
_RPW = 624
_RTAIL = _N - 16 * _RPW


def _mesh():
    return plsc.VectorSubcoreMesh(core_axis_name="c", subcore_axis_name="s")


def _fill(buf, nrows, val):
    v = jnp.full((16,), val, jnp.float32)

    def row(i, carry):
        for j in range(_W // 16):
            buf[i, pl.ds(j * 16, 16)] = v
        return carry

    lax.fori_loop(0, nrows, row, None)


def _zero_acc(zbuf, acc, s):
    for jj in range(_RPW // _ZR):
        off = pl.multiple_of(s * _RPW + jj * _ZR, 8)
        pltpu.sync_copy(zbuf, acc.at[pl.ds(off, _ZR)])

    @pl.when(s == 15)
    def _():
        pltpu.sync_copy(zbuf.at[pl.ds(0, _RTAIL)],
                        acc.at[pl.ds(16 * _RPW, _RTAIL)])


def _writeback(acc, out_hbm, c, s):
    off = pl.multiple_of(s * _RPW, 8)
    dst_off = pl.multiple_of(c * _N + s * _RPW, 8)
    pltpu.sync_copy(acc.at[pl.ds(off, _RPW)], out_hbm.at[pl.ds(dst_off, _RPW)])

    @pl.when(s == 15)
    def _():
        doff = pl.multiple_of(c * _N + 16 * _RPW, 8)
        pltpu.sync_copy(acc.at[pl.ds(16 * _RPW, _RTAIL)],
                        out_hbm.at[pl.ds(doff, _RTAIL)])


def _make_deg():
    epw = _E // 32
    nch = epw // _CH
    tail = epw - nch * _CH

    scratch = [
        pltpu.VMEM((_CH,), jnp.int32),
        pltpu.VMEM((_CH, _W), jnp.float32),
        pltpu.VMEM((tail,), jnp.int32),
        pltpu.VMEM((_ZR, _W), jnp.float32),
        pltpu.VMEM_SHARED((_N, _W), jnp.float32),
    ]

    @functools.partial(
        pl.kernel,
        mesh=_mesh(),
        out_type=jax.ShapeDtypeStruct((2 * _N, _W), jnp.float32),
        scratch_types=scratch,
    )
    def deg_kernel(dst_hbm, out_hbm, idx_d, ones, idx_t, zbuf, acc):
        c = lax.axis_index("c")
        s = lax.axis_index("s")
        _fill(zbuf, _ZR, 0.0)
        _fill(ones, _CH, 1.0)
        _zero_acc(zbuf, acc, s)
        plsc.subcore_barrier()

        base = (c * 16 + s) * epw

        def chunk(k, carry):
            off = pl.multiple_of(base + k * _CH, 8)
            pltpu.sync_copy(dst_hbm.at[pl.ds(off, _CH)], idx_d)
            pltpu.sync_copy(ones, acc.at[idx_d], add=True)
            return carry

        lax.fori_loop(0, nch, chunk, None)
        if tail:
            off = pl.multiple_of(base + nch * _CH, 8)
            pltpu.sync_copy(dst_hbm.at[pl.ds(off, tail)], idx_t)
            pltpu.sync_copy(ones.at[pl.ds(0, tail)], acc.at[idx_t], add=True)
        plsc.subcore_barrier()
        _writeback(acc, out_hbm, c, s)

    return deg_kernel


def _make_agg(feature_split):
    epw = _E // 16 if feature_split else _E // 32
    nch = epw // _CH
    tail = epw - nch * _CH

    scratch = [
        pltpu.VMEM((_CH,), jnp.int32),
        pltpu.VMEM((_CH,), jnp.int32),
        pltpu.VMEM((_CH, _W), jnp.float32),
        pltpu.VMEM((tail,), jnp.int32),
        pltpu.VMEM((tail,), jnp.int32),
        pltpu.VMEM((tail, _W), jnp.float32),
        pltpu.VMEM((_ZR, _W), jnp.float32),
        pltpu.VMEM_SHARED((_N, _W), jnp.float32),
        pltpu.SemaphoreType.DMA,
    ]

    @functools.partial(
        pl.kernel,
        mesh=_mesh(),
        out_type=jax.ShapeDtypeStruct((2 * _N, _W), jnp.float32),
        scratch_types=scratch,
    )
    def agg_kernel(h_hbm, src_hbm, dst_hbm, out_hbm,
                   idx_s, idx_d, rows, idx_ts, idx_td, rows_t, zbuf, acc, sem):
        c = lax.axis_index("c")
        s = lax.axis_index("s")
        _fill(zbuf, _ZR, 0.0)
        _zero_acc(zbuf, acc, s)
        plsc.subcore_barrier()

        if feature_split:
            base = s * epw
            coff = c * _N
        else:
            base = (c * 16 + s) * epw
            coff = None

        def body(off, ibs, ibd, rbuf, width):
            pltpu.sync_copy(src_hbm.at[pl.ds(off, width)], ibs)
            pltpu.sync_copy(dst_hbm.at[pl.ds(off, width)], ibd)
            if feature_split:
                for j in range(width // 16):
                    ibs[pl.ds(j * 16, 16)] = ibs[pl.ds(j * 16, 16)] + coff
            pltpu.async_copy(h_hbm.at[ibs], rbuf, sem).wait()
            pltpu.sync_copy(rbuf, acc.at[ibd], add=True)

        def chunk(k, carry):
            body(pl.multiple_of(base + k * _CH, 8), idx_s, idx_d, rows, _CH)
            return carry

        lax.fori_loop(0, nch, chunk, None)
        if tail:
            body(pl.multiple_of(base + nch * _CH, 8),
                 idx_ts, idx_td, rows_t, tail)
        plsc.subcore_barrier()
        _writeback(acc, out_hbm, c, s)

    return agg_kernel


def _dinv_from(degp):
    deg = degp[0][:, 0] + degp[1][:, 0] + 1.0
    return lax.rsqrt(deg)[:, None]


def _tc_a_body(x_ref, w_ref, degp_ref, out_ref):
    dinv = _dinv_from(degp_ref)
    h = jnp.dot(x_ref[...], w_ref[...], preferred_element_type=jnp.float32)
    hs = h * dinv
    out_ref[0] = hs[:, : _DH // 2]
    out_ref[1] = hs[:, _DH // 2:]


def _tc_b_body(agg_ref, hs_ref, degp_ref, b1_ref, w2_ref, out_ref):
    dinv = _dinv_from(degp_ref)
    a = jnp.concatenate([agg_ref[0] + hs_ref[0], agg_ref[1] + hs_ref[1]], axis=1)
    z = jnp.maximum(a * dinv + b1_ref[...], 0.0)
    h2 = jnp.dot(z, w2_ref[...], preferred_element_type=jnp.float32) * dinv
    out_ref[...] = jnp.concatenate(
        [h2, jnp.zeros((h2.shape[0], _W - _DOUT), jnp.float32)], axis=1)


def _tc_c_body(agg_ref, hs_ref, degp_ref, b2_ref, out_ref):
    dinv = _dinv_from(degp_ref)
    z = (agg_ref[0][:, : _DOUT] + agg_ref[1][:, : _DOUT]
         + hs_ref[:, : _DOUT]) * dinv + b2_ref[...]
    m = jnp.max(z, axis=1, keepdims=True)
    lse = jnp.log(jnp.sum(jnp.exp(z - m), axis=1, keepdims=True)) + m
    out_ref[...] = z - lse


def _node_spec(width):
    return pl.BlockSpec((_BN, width), lambda n: (n, 0))


def _split_spec(width):
    return pl.BlockSpec((2, _BN, width), lambda n: (0, n, 0))


def _full_spec(r, c):
    return pl.BlockSpec((r, c), lambda n: (0, 0))


def _tc_a(x, w1, degp):
    return pl.pallas_call(
        _tc_a_body,
        grid=(_N // _BN,),
        in_specs=[_node_spec(_DIN), _full_spec(_DIN, _DH), _split_spec(_W)],
        out_specs=_split_spec(_DH // 2),
        out_shape=jax.ShapeDtypeStruct((2, _N, _DH // 2), jnp.float32),
    )(x, w1, degp)


def _tc_b(agg1, hs1, degp, b1, w2):
    return pl.pallas_call(
        _tc_b_body,
        grid=(_N // _BN,),
        in_specs=[_split_spec(_DH // 2), _split_spec(_DH // 2),
                  _split_spec(_W), _full_spec(1, _DH), _full_spec(_DH, _DOUT)],
        out_specs=_node_spec(_W),
        out_shape=jax.ShapeDtypeStruct((_N, _W), jnp.float32),
    )(agg1, hs1, degp, b1, w2)


def _tc_c(agg2, hs2, degp, b2):
    return pl.pallas_call(
        _tc_c_body,
        grid=(_N // _BN,),
        in_specs=[_split_spec(_W), _node_spec(_W),
                  _split_spec(_W), _full_spec(1, _DOUT)],
        out_specs=_node_spec(_DOUT),
        out_shape=jax.ShapeDtypeStruct((_N, _DOUT), jnp.float32),
    )(agg2, hs2, degp, b2)


def kernel(x, edge_index, W1, b1, W2, b2):
    src = edge_index[0]
    dst = edge_index[1]

    degp = _make_deg()(dst).reshape(2, _N, _W)

    hs1 = _tc_a(x, W1, degp)
    agg1 = _make_agg(True)(hs1.reshape(2 * _N, _DH // 2), src, dst)
    agg1 = agg1.reshape(2, _N, _DH // 2)

    hs2 = _tc_b(agg1, hs1, degp, b1.reshape(1, _DH), W2)
    agg2 = _make_agg(False)(hs2, src, dst).reshape(2, _N, _W)

    return _tc_c(agg2, hs2, degp, b2.reshape(1, _DOUT))


# speedup vs baseline: 19.1710x; 1.7728x over previous
import functools

import jax
import jax.numpy as jnp
from jax import lax
from jax.experimental import pallas as pl
from jax.experimental.pallas import tpu as pltpu
from jax.experimental.pallas import tpu_sc as plsc

_N = 10000
_E = 160000
_DIN = 256
_DH = 256
_DOUT = 64
_W = 128
_BN = 1000
_CH = 128
_NR = _E // _CH
_NG = _NR // 8
_XROW = _NR - 8 * _NG

_ZR = 48
_RPW = 624
_RTAIL = _N - 16 * _RPW


def _mesh():
    return plsc.VectorSubcoreMesh(core_axis_name="c", subcore_axis_name="s")


def _zero_rows(buf, nrows, width):
    z = jnp.zeros((16,), jnp.float32)

    def row(i, carry):
        for j in range(width // 16):
            buf[i, pl.ds(j * 16, 16)] = z
        return carry

    lax.fori_loop(0, nrows, row, None)


def _zero_acc(zbuf, acc, s):
    for jj in range(_RPW // _ZR):
        off = pl.multiple_of(s * _RPW + jj * _ZR, 8)
        pltpu.sync_copy(zbuf, acc.at[pl.ds(off, _ZR)])

    @pl.when(s == 15)
    def _():
        pltpu.sync_copy(zbuf.at[pl.ds(0, _RTAIL)],
                        acc.at[pl.ds(16 * _RPW, _RTAIL)])


def _writeback(acc, out_hbm, c, s):
    off = pl.multiple_of(s * _RPW, 8)
    dst_off = pl.multiple_of(c * _N + s * _RPW, 8)
    pltpu.sync_copy(acc.at[pl.ds(off, _RPW)], out_hbm.at[pl.ds(dst_off, _RPW)])

    @pl.when(s == 15)
    def _():
        doff = pl.multiple_of(c * _N + 16 * _RPW, 8)
        pltpu.sync_copy(acc.at[pl.ds(16 * _RPW, _RTAIL)],
                        out_hbm.at[pl.ds(doff, _RTAIL)])


def _make_deg():
    split = 32
    ng_even = _NG // split
    rem_g = _NG - ng_even * split
    nc_cap = 8 * (ng_even + 1) + 2

    scratch = [
        pltpu.VMEM((nc_cap, _CH), jnp.int32),
        pltpu.VMEM((_CH, _W), jnp.float32),
        pltpu.VMEM((_ZR, _W), jnp.float32),
        pltpu.VMEM_SHARED((_N, _W), jnp.float32),
        pltpu.SemaphoreType.DMA,
    ]

    @functools.partial(
        pl.kernel,
        mesh=_mesh(),
        out_type=jax.ShapeDtypeStruct((2 * _N, _W), jnp.float32),
        scratch_types=scratch,
    )
    def deg_kernel(ei_hbm, out_hbm, ib, ones, zbuf, acc, sem):
        c = lax.axis_index("c")
        s = lax.axis_index("s")
        w = c * 16 + s

        _zero_rows(zbuf, _ZR, _W)
        one = jnp.ones((16,), jnp.float32)

        def orow(i, carry):
            for j in range(_W // 16):
                ones[i, pl.ds(j * 16, 16)] = one
            return carry

        lax.fori_loop(0, _CH, orow, None)
        _zero_acc(zbuf, acc, s)

        ng = ng_even + jnp.where(w < rem_g, 1, 0)
        nc = pl.multiple_of(ng * 8, 8)
        has_extra = w == rem_g

        def ldgroup(j, carry):
            rbase = pl.multiple_of(8 * (split * j + w), 8)
            dbase = pl.multiple_of(8 * j, 8)
            pltpu.sync_copy(ei_hbm.at[1, pl.ds(rbase, 8), :],
                            ib.at[pl.ds(dbase, 8), :])
            return carry

        lax.fori_loop(0, ng, ldgroup, None)

        @pl.when(has_extra)
        def _():
            pltpu.sync_copy(ei_hbm.at[1, pl.ds(8 * _NG, _XROW), :],
                            ib.at[pl.ds(nc, _XROW), :])

        plsc.subcore_barrier()

        ntot = nc + jnp.where(has_extra, _XROW, 0)

        def fire(k, carry):
            pltpu.async_copy(ones, acc.at[ib.at[k]], sem, add=True)
            return carry

        lax.fori_loop(0, ntot, fire, None)

        def drain(k, carry):
            pltpu.make_async_copy(ones, acc.at[ib.at[0]], sem).wait()
            return carry

        lax.fori_loop(0, ntot, drain, None)

        plsc.subcore_barrier()
        _writeback(acc, out_hbm, c, s)

    return deg_kernel


def _make_agg(feature_split):
    split = 16 if feature_split else 32
    ng_even = _NG // split
    rem_g = _NG - ng_even * split

    scratch = (
        [pltpu.VMEM((2, _CH), jnp.int32)] * 4 +
        [pltpu.VMEM((_CH, _W), jnp.float32)] * 2 +
        [pltpu.VMEM((_ZR, _W), jnp.float32),
         pltpu.VMEM_SHARED((_N, _W), jnp.float32)] +
        [pltpu.SemaphoreType.DMA] * 8
    )

    @functools.partial(
        pl.kernel,
        mesh=_mesh(),
        out_type=jax.ShapeDtypeStruct((2 * _N, _W), jnp.float32),
        scratch_types=scratch,
    )
    def agg_kernel(ei_hbm, h_hbm, out_hbm, ib0, ib1, ib2, ib3, r0, r1,
                   zbuf, acc, si0, si1, si2, si3, sg0, sg1, ss0, ss1):
        c = lax.axis_index("c")
        s = lax.axis_index("s")
        w = s if feature_split else c * 16 + s
        ibq = [ib0, ib1, ib2, ib3]
        sis = [si0, si1, si2, si3]
        rs = [r0, r1]
        sgs = [sg0, sg1]
        sss = [ss0, ss1]
        coff = c * _N

        _zero_rows(zbuf, _ZR, _W)
        _zero_acc(zbuf, acc, s)

        ng = ng_even + jnp.where(w < rem_g, 1, 0)
        nc = ng * 8
        has_extra = w == rem_g

        def ei_off(k):
            j = k // 8
            return pl.multiple_of((split * j + w) * 1024 + (k - 8 * j) * 128,
                                  128)

        def i_start(k, q):
            pltpu.async_copy(ei_hbm.at[:, pl.ds(ei_off(k), _CH)],
                             ibq[q], sis[q])

        def i_wait(k, q):
            pltpu.make_async_copy(ei_hbm.at[:, pl.ds(ei_off(k), _CH)],
                                  ibq[q], sis[q]).wait()

        def adj(q):
            if feature_split:
                for m in range(_CH // 16):
                    ibq[q][0, pl.ds(m * 16, 16)] = (
                        ibq[q][0, pl.ds(m * 16, 16)] + coff)

        def g_start(p, q):
            pltpu.async_copy(h_hbm.at[ibq[q].at[0]], rs[p], sgs[p])

        def g_wait(p, q):
            pltpu.make_async_copy(h_hbm.at[ibq[q].at[0]], rs[p], sgs[p]).wait()

        def s_start(p, q):
            pltpu.async_copy(rs[p], acc.at[ibq[q].at[1]], sss[p], add=True)

        def s_wait(p, q):
            pltpu.make_async_copy(rs[p], acc.at[ibq[q].at[1]], sss[p]).wait()

        plsc.subcore_barrier()

        i_start(0, 0)
        i_start(1, 1)
        i_start(2, 2)
        i_wait(0, 0)
        adj(0)
        g_start(0, 0)
        i_start(3, 3)
        i_wait(1, 1)
        adj(1)
        g_start(1, 1)
        g_wait(0, 0)
        s_start(0, 0)
        s_wait(0, 2)
        i_start(4, 0)
        i_wait(2, 2)
        adj(2)
        g_start(0, 2)
        g_wait(1, 1)
        s_start(1, 1)
        s_wait(1, 3)
        i_start(5, 1)
        i_wait(3, 3)
        adj(3)
        g_start(1, 3)
        g_wait(0, 2)
        s_start(0, 2)

        def quad(k4, carry):
            for q in range(4):
                k = k4 * 4 + q
                p = q % 2
                s_wait(p, q)

                @pl.when(k + 2 < nc)
                def _():
                    i_start(k + 2, (q + 2) % 4)

                i_wait(k, q)
                adj(q)
                g_start(p, q)
                g_wait(1 - p, (q + 3) % 4)
                s_start(1 - p, (q + 3) % 4)
            return carry

        lax.fori_loop(1, nc // 4, quad, None)

        g_wait(1, 3)
        s_start(1, 3)
        s_wait(0, 2)
        s_wait(1, 3)

        @pl.when(has_extra)
        def _():
            for j in range(_XROW):
                off = (8 * _NG + j) * 128
                pltpu.sync_copy(ei_hbm.at[:, pl.ds(off, _CH)], ibq[j])
                adj(j)
                pltpu.async_copy(h_hbm.at[ibq[j].at[0]], rs[j], sgs[j]).wait()
                pltpu.async_copy(rs[j], acc.at[ibq[j].at[1]], sss[j],
                                 add=True).wait()

        plsc.subcore_barrier()
        _writeback(acc, out_hbm, c, s)

    return agg_kernel


def _tc_d_body(degp_ref, out_ref):
    deg = degp_ref[0][:, 0] + degp_ref[1][:, 0] + 1.0
    out_ref[...] = lax.rsqrt(deg)[:, None]


def _tc_a_body(x_ref, w_ref, dinv_ref, out_ref):
    dinv = dinv_ref[...]
    h = jnp.dot(x_ref[...], w_ref[...], preferred_element_type=jnp.float32)
    hs = h * dinv
    out_ref[0] = hs[:, : _DH // 2]
    out_ref[1] = hs[:, _DH // 2:]


def _tc_b_body(agg_ref, hs_ref, dinv_ref, b1_ref, w2_ref, out_ref):
    dinv = dinv_ref[...]
    a = jnp.concatenate([agg_ref[0] + hs_ref[0], agg_ref[1] + hs_ref[1]], axis=1)
    z = jnp.maximum(a * dinv + b1_ref[...], 0.0)
    h2 = jnp.dot(z, w2_ref[...], preferred_element_type=jnp.float32) * dinv
    out_ref[...] = jnp.concatenate(
        [h2, jnp.zeros((h2.shape[0], _W - _DOUT), jnp.float32)], axis=1)


def _tc_c_body(agg_ref, hs_ref, dinv_ref, b2_ref, out_ref):
    dinv = dinv_ref[...]
    z = (agg_ref[0][:, : _DOUT] + agg_ref[1][:, : _DOUT]
         + hs_ref[:, : _DOUT]) * dinv + b2_ref[...]
    m = jnp.max(z, axis=1, keepdims=True)
    lse = jnp.log(jnp.sum(jnp.exp(z - m), axis=1, keepdims=True)) + m
    out_ref[...] = z - lse


def _node_spec(width):
    return pl.BlockSpec((_BN, width), lambda n: (n, 0))


def _split_spec(width):
    return pl.BlockSpec((2, _BN, width), lambda n: (0, n, 0))


def _dinv_spec():
    return pl.BlockSpec((_BN, 1), lambda n: (n, 0))


def _full_spec(r, c):
    return pl.BlockSpec((r, c), lambda n: (0, 0))


def _tc_d(degp):
    return pl.pallas_call(
        _tc_d_body,
        grid=(_N // _BN,),
        in_specs=[_split_spec(_W)],
        out_specs=_dinv_spec(),
        out_shape=jax.ShapeDtypeStruct((_N, 1), jnp.float32),
    )(degp)


def _tc_a(x, w1, dinv):
    return pl.pallas_call(
        _tc_a_body,
        grid=(_N // _BN,),
        in_specs=[_node_spec(_DIN), _full_spec(_DIN, _DH), _dinv_spec()],
        out_specs=_split_spec(_DH // 2),
        out_shape=jax.ShapeDtypeStruct((2, _N, _DH // 2), jnp.float32),
    )(x, w1, dinv)


def _tc_b(agg1, hs1, dinv, b1, w2):
    return pl.pallas_call(
        _tc_b_body,
        grid=(_N // _BN,),
        in_specs=[_split_spec(_DH // 2), _split_spec(_DH // 2),
                  _dinv_spec(), _full_spec(1, _DH), _full_spec(_DH, _DOUT)],
        out_specs=_node_spec(_W),
        out_shape=jax.ShapeDtypeStruct((_N, _W), jnp.float32),
    )(agg1, hs1, dinv, b1, w2)


def _tc_c(agg2, hs2, dinv, b2):
    return pl.pallas_call(
        _tc_c_body,
        grid=(_N // _BN,),
        in_specs=[_split_spec(_W), _node_spec(_W),
                  _dinv_spec(), _full_spec(1, _DOUT)],
        out_specs=_node_spec(_DOUT),
        out_shape=jax.ShapeDtypeStruct((_N, _DOUT), jnp.float32),
    )(agg2, hs2, dinv, b2)


def kernel(x, edge_index, W1, b1, W2, b2):
    ei3 = edge_index.reshape(2, _NR, _CH)

    degp = _make_deg()(ei3).reshape(2, _N, _W)
    dinv = _tc_d(degp)

    hs1 = _tc_a(x, W1, dinv)
    agg1 = _make_agg(True)(edge_index, hs1.reshape(2 * _N, _DH // 2))
    agg1 = agg1.reshape(2, _N, _DH // 2)

    hs2 = _tc_b(agg1, hs1, dinv, b1.reshape(1, _DH), W2)
    agg2 = _make_agg(False)(edge_index, hs2).reshape(2, _N, _W)

    return _tc_c(agg2, hs2, dinv, b2.reshape(1, _DOUT))


# speedup vs baseline: 20.9301x; 1.0918x over previous
import functools

import jax
import jax.numpy as jnp
from jax import lax
from jax.experimental import pallas as pl
from jax.experimental.pallas import tpu as pltpu
from jax.experimental.pallas import tpu_sc as plsc

_N = 10000
_E = 160000
_DIN = 256
_DH = 256
_DOUT = 64
_W = 128
_BN = 1000
_CH = 128
_NR = _E // _CH
_NG = _NR // 8
_XROW = _NR - 8 * _NG

_ZR = 48
_RPW = 624
_RTAIL = _N - 16 * _RPW


def _mesh():
    return plsc.VectorSubcoreMesh(core_axis_name="c", subcore_axis_name="s")


def _zero_rows(buf, nrows, width):
    z = jnp.zeros((16,), jnp.float32)

    def row(i, carry):
        for j in range(width // 16):
            buf[i, pl.ds(j * 16, 16)] = z
        return carry

    lax.fori_loop(0, nrows, row, None)


def _zero_acc(zbuf, acc, s):
    for jj in range(_RPW // _ZR):
        off = pl.multiple_of(s * _RPW + jj * _ZR, 8)
        pltpu.sync_copy(zbuf, acc.at[pl.ds(off, _ZR)])

    @pl.when(s == 15)
    def _():
        pltpu.sync_copy(zbuf.at[pl.ds(0, _RTAIL)],
                        acc.at[pl.ds(16 * _RPW, _RTAIL)])


def _writeback(acc, out_hbm, c, s):
    off = pl.multiple_of(s * _RPW, 8)
    dst_off = pl.multiple_of(c * _N + s * _RPW, 8)
    pltpu.sync_copy(acc.at[pl.ds(off, _RPW)], out_hbm.at[pl.ds(dst_off, _RPW)])

    @pl.when(s == 15)
    def _():
        doff = pl.multiple_of(c * _N + 16 * _RPW, 8)
        pltpu.sync_copy(acc.at[pl.ds(16 * _RPW, _RTAIL)],
                        out_hbm.at[pl.ds(doff, _RTAIL)])


_DW = 16


def _make_deg():
    split = 32
    ng_even = _NG // split
    rem_g = _NG - ng_even * split
    nc_cap = 8 * (ng_even + 1) + 2

    scratch = [
        pltpu.VMEM((nc_cap, _CH), jnp.int32),
        pltpu.VMEM((_CH, _DW), jnp.float32),
        pltpu.VMEM((_ZR, _DW), jnp.float32),
        pltpu.VMEM_SHARED((_N, _DW), jnp.float32),
        pltpu.SemaphoreType.DMA,
    ]

    @functools.partial(
        pl.kernel,
        mesh=_mesh(),
        out_type=jax.ShapeDtypeStruct((2 * _N, _DW), jnp.float32),
        scratch_types=scratch,
    )
    def deg_kernel(ei_hbm, out_hbm, ib, ones, zbuf, acc, sem):
        c = lax.axis_index("c")
        s = lax.axis_index("s")
        w = c * 16 + s

        _zero_rows(zbuf, _ZR, _DW)
        one = jnp.ones((16,), jnp.float32)

        def orow(i, carry):
            for j in range(_DW // 16):
                ones[i, pl.ds(j * 16, 16)] = one
            return carry

        lax.fori_loop(0, _CH, orow, None)
        _zero_acc(zbuf, acc, s)

        ng = ng_even + jnp.where(w < rem_g, 1, 0)
        nc = pl.multiple_of(ng * 8, 8)
        has_extra = w == rem_g

        def ldgroup(j, carry):
            rbase = pl.multiple_of(8 * (split * j + w), 8)
            dbase = pl.multiple_of(8 * j, 8)
            pltpu.sync_copy(ei_hbm.at[1, pl.ds(rbase, 8), :],
                            ib.at[pl.ds(dbase, 8), :])
            return carry

        lax.fori_loop(0, ng, ldgroup, None)

        @pl.when(has_extra)
        def _():
            pltpu.sync_copy(ei_hbm.at[1, pl.ds(8 * _NG, _XROW), :],
                            ib.at[pl.ds(nc, _XROW), :])

        plsc.subcore_barrier()

        ntot = nc + jnp.where(has_extra, _XROW, 0)

        def fire(k, carry):
            pltpu.async_copy(ones, acc.at[ib.at[k]], sem, add=True)
            return carry

        lax.fori_loop(0, ntot, fire, None)

        def drain(k, carry):
            pltpu.make_async_copy(ones, acc.at[ib.at[0]], sem).wait()
            return carry

        lax.fori_loop(0, ntot, drain, None)

        plsc.subcore_barrier()
        _writeback(acc, out_hbm, c, s)

    return deg_kernel


def _make_agg(feature_split):
    split = 16 if feature_split else 32
    ng_even = _NG // split
    rem_g = _NG - ng_even * split

    scratch = (
        [pltpu.VMEM((2, _CH), jnp.int32)] * 4 +
        [pltpu.VMEM((_CH, _W), jnp.float32)] * 2 +
        [pltpu.VMEM((_ZR, _W), jnp.float32),
         pltpu.VMEM_SHARED((_N, _W), jnp.float32)] +
        [pltpu.SemaphoreType.DMA] * 8
    )

    @functools.partial(
        pl.kernel,
        mesh=_mesh(),
        out_type=jax.ShapeDtypeStruct((2 * _N, _W), jnp.float32),
        scratch_types=scratch,
    )
    def agg_kernel(ei_hbm, h_hbm, out_hbm, ib0, ib1, ib2, ib3, r0, r1,
                   zbuf, acc, si0, si1, si2, si3, sg0, sg1, ss0, ss1):
        c = lax.axis_index("c")
        s = lax.axis_index("s")
        w = s if feature_split else c * 16 + s
        ibq = [ib0, ib1, ib2, ib3]
        sis = [si0, si1, si2, si3]
        rs = [r0, r1]
        sgs = [sg0, sg1]
        sss = [ss0, ss1]
        coff = c * _N

        _zero_rows(zbuf, _ZR, _W)
        _zero_acc(zbuf, acc, s)

        ng = ng_even + jnp.where(w < rem_g, 1, 0)
        nc = ng * 8
        has_extra = w == rem_g

        def ei_off(k):
            j = k // 8
            return pl.multiple_of((split * j + w) * 1024 + (k - 8 * j) * 128,
                                  128)

        def i_start(k, q):
            pltpu.async_copy(ei_hbm.at[:, pl.ds(ei_off(k), _CH)],
                             ibq[q], sis[q])

        def i_wait(k, q):
            pltpu.make_async_copy(ei_hbm.at[:, pl.ds(ei_off(k), _CH)],
                                  ibq[q], sis[q]).wait()

        def adj(q):
            if feature_split:
                for m in range(_CH // 16):
                    ibq[q][0, pl.ds(m * 16, 16)] = (
                        ibq[q][0, pl.ds(m * 16, 16)] + coff)

        def g_start(p, q):
            pltpu.async_copy(h_hbm.at[ibq[q].at[0]], rs[p], sgs[p])

        def g_wait(p, q):
            pltpu.make_async_copy(h_hbm.at[ibq[q].at[0]], rs[p], sgs[p]).wait()

        def s_start(p, q):
            pltpu.async_copy(rs[p], acc.at[ibq[q].at[1]], sss[p], add=True)

        def s_wait(p, q):
            pltpu.make_async_copy(rs[p], acc.at[ibq[q].at[1]], sss[p]).wait()

        plsc.subcore_barrier()

        i_start(0, 0)
        i_start(1, 1)
        i_start(2, 2)
        i_wait(0, 0)
        adj(0)
        g_start(0, 0)
        i_start(3, 3)
        i_wait(1, 1)
        adj(1)
        g_start(1, 1)
        g_wait(0, 0)
        s_start(0, 0)
        s_wait(0, 2)
        i_start(4, 0)
        i_wait(2, 2)
        adj(2)
        g_start(0, 2)
        g_wait(1, 1)
        s_start(1, 1)
        s_wait(1, 3)
        i_start(5, 1)
        i_wait(3, 3)
        adj(3)
        g_start(1, 3)
        g_wait(0, 2)
        s_start(0, 2)

        def quad(k4, carry):
            for q in range(4):
                k = k4 * 4 + q
                p = q % 2
                s_wait(p, q)

                @pl.when(k + 2 < nc)
                def _():
                    i_start(k + 2, (q + 2) % 4)

                i_wait(k, q)
                adj(q)
                g_start(p, q)
                g_wait(1 - p, (q + 3) % 4)
                s_start(1 - p, (q + 3) % 4)
            return carry

        lax.fori_loop(1, nc // 4, quad, None)

        g_wait(1, 3)
        s_start(1, 3)
        s_wait(0, 2)
        s_wait(1, 3)

        @pl.when(has_extra)
        def _():
            for j in range(_XROW):
                off = (8 * _NG + j) * 128
                pltpu.sync_copy(ei_hbm.at[:, pl.ds(off, _CH)], ibq[j])
                adj(j)
                pltpu.async_copy(h_hbm.at[ibq[j].at[0]], rs[j], sgs[j]).wait()
                pltpu.async_copy(rs[j], acc.at[ibq[j].at[1]], sss[j],
                                 add=True).wait()

        plsc.subcore_barrier()
        _writeback(acc, out_hbm, c, s)

    return agg_kernel


def _tc_d_body(degp_ref, out_ref):
    deg = degp_ref[0][:, 0] + degp_ref[1][:, 0] + 1.0
    out_ref[...] = lax.rsqrt(deg)[:, None]


def _tc_a_body(x_ref, w_ref, dinv_ref, out_ref):
    dinv = dinv_ref[...]
    h = jnp.dot(x_ref[...], w_ref[...], preferred_element_type=jnp.float32)
    hs = h * dinv
    out_ref[0] = hs[:, : _DH // 2]
    out_ref[1] = hs[:, _DH // 2:]


def _tc_b_body(agg_ref, hs_ref, dinv_ref, b1_ref, w2_ref, out_ref):
    dinv = dinv_ref[...]
    a = jnp.concatenate([agg_ref[0] + hs_ref[0], agg_ref[1] + hs_ref[1]], axis=1)
    z = jnp.maximum(a * dinv + b1_ref[...], 0.0)
    h2 = jnp.dot(z, w2_ref[...], preferred_element_type=jnp.float32) * dinv
    out_ref[...] = jnp.concatenate(
        [h2, jnp.zeros((h2.shape[0], _W - _DOUT), jnp.float32)], axis=1)


def _tc_c_body(agg_ref, hs_ref, dinv_ref, b2_ref, out_ref):
    dinv = dinv_ref[...]
    z = (agg_ref[0][:, : _DOUT] + agg_ref[1][:, : _DOUT]
         + hs_ref[:, : _DOUT]) * dinv + b2_ref[...]
    m = jnp.max(z, axis=1, keepdims=True)
    lse = jnp.log(jnp.sum(jnp.exp(z - m), axis=1, keepdims=True)) + m
    out_ref[...] = z - lse


def _node_spec(width):
    return pl.BlockSpec((_BN, width), lambda n: (n, 0))


def _split_spec(width):
    return pl.BlockSpec((2, _BN, width), lambda n: (0, n, 0))


def _dinv_spec():
    return pl.BlockSpec((_BN, 1), lambda n: (n, 0))


def _full_spec(r, c):
    return pl.BlockSpec((r, c), lambda n: (0, 0))


def _tc_d(degp):
    return pl.pallas_call(
        _tc_d_body,
        grid=(_N // _BN,),
        in_specs=[_split_spec(_DW)],
        out_specs=_dinv_spec(),
        out_shape=jax.ShapeDtypeStruct((_N, 1), jnp.float32),
    )(degp)


def _tc_a(x, w1, dinv):
    return pl.pallas_call(
        _tc_a_body,
        grid=(_N // _BN,),
        in_specs=[_node_spec(_DIN), _full_spec(_DIN, _DH), _dinv_spec()],
        out_specs=_split_spec(_DH // 2),
        out_shape=jax.ShapeDtypeStruct((2, _N, _DH // 2), jnp.float32),
    )(x, w1, dinv)


def _tc_b(agg1, hs1, dinv, b1, w2):
    return pl.pallas_call(
        _tc_b_body,
        grid=(_N // _BN,),
        in_specs=[_split_spec(_DH // 2), _split_spec(_DH // 2),
                  _dinv_spec(), _full_spec(1, _DH), _full_spec(_DH, _DOUT)],
        out_specs=_node_spec(_W),
        out_shape=jax.ShapeDtypeStruct((_N, _W), jnp.float32),
    )(agg1, hs1, dinv, b1, w2)


def _tc_c(agg2, hs2, dinv, b2):
    return pl.pallas_call(
        _tc_c_body,
        grid=(_N // _BN,),
        in_specs=[_split_spec(_W), _node_spec(_W),
                  _dinv_spec(), _full_spec(1, _DOUT)],
        out_specs=_node_spec(_DOUT),
        out_shape=jax.ShapeDtypeStruct((_N, _DOUT), jnp.float32),
    )(agg2, hs2, dinv, b2)


def kernel(x, edge_index, W1, b1, W2, b2):
    ei3 = edge_index.reshape(2, _NR, _CH)

    degp = _make_deg()(ei3).reshape(2, _N, _DW)
    dinv = _tc_d(degp)

    hs1 = _tc_a(x, W1, dinv)
    agg1 = _make_agg(True)(edge_index, hs1.reshape(2 * _N, _DH // 2))
    agg1 = agg1.reshape(2, _N, _DH // 2)

    hs2 = _tc_b(agg1, hs1, dinv, b1.reshape(1, _DH), W2)
    agg2 = _make_agg(False)(edge_index, hs2).reshape(2, _N, _W)

    return _tc_c(agg2, hs2, dinv, b2.reshape(1, _DOUT))


# speedup vs baseline: 22.0540x; 1.0537x over previous
import functools

import jax
import jax.numpy as jnp
from jax import lax
from jax.experimental import pallas as pl
from jax.experimental.pallas import tpu as pltpu
from jax.experimental.pallas import tpu_sc as plsc

_N = 10000
_E = 160000
_DIN = 256
_DH = 256
_DOUT = 64
_W = 128
_BN = 2000
_CH = 128
_NR = _E // _CH
_NG = _NR // 8
_XROW = _NR - 8 * _NG

_ZR = 48
_RPW = 624
_RTAIL = _N - 16 * _RPW


def _mesh():
    return plsc.VectorSubcoreMesh(core_axis_name="c", subcore_axis_name="s")


def _zero_rows(buf, nrows, width):
    z = jnp.zeros((16,), jnp.float32)

    def row(i, carry):
        for j in range(width // 16):
            buf[i, pl.ds(j * 16, 16)] = z
        return carry

    lax.fori_loop(0, nrows, row, None)


def _zero_acc(zbuf, acc, s):
    for jj in range(_RPW // _ZR):
        off = pl.multiple_of(s * _RPW + jj * _ZR, 8)
        pltpu.sync_copy(zbuf, acc.at[pl.ds(off, _ZR)])

    @pl.when(s == 15)
    def _():
        pltpu.sync_copy(zbuf.at[pl.ds(0, _RTAIL)],
                        acc.at[pl.ds(16 * _RPW, _RTAIL)])


def _writeback(acc, out_hbm, c, s):
    off = pl.multiple_of(s * _RPW, 8)
    dst_off = pl.multiple_of(c * _N + s * _RPW, 8)
    pltpu.sync_copy(acc.at[pl.ds(off, _RPW)], out_hbm.at[pl.ds(dst_off, _RPW)])

    @pl.when(s == 15)
    def _():
        doff = pl.multiple_of(c * _N + 16 * _RPW, 8)
        pltpu.sync_copy(acc.at[pl.ds(16 * _RPW, _RTAIL)],
                        out_hbm.at[pl.ds(doff, _RTAIL)])


_DW = 16


def _make_deg():
    split = 32
    ng_even = _NG // split
    rem_g = _NG - ng_even * split
    nc_cap = 8 * (ng_even + 1) + 2

    scratch = [
        pltpu.VMEM((nc_cap, _CH), jnp.int32),
        pltpu.VMEM((_CH, _DW), jnp.float32),
        pltpu.VMEM((_ZR, _DW), jnp.float32),
        pltpu.VMEM_SHARED((_N, _DW), jnp.float32),
        pltpu.SemaphoreType.DMA,
    ]

    @functools.partial(
        pl.kernel,
        mesh=_mesh(),
        out_type=jax.ShapeDtypeStruct((2 * _N, _DW), jnp.float32),
        scratch_types=scratch,
    )
    def deg_kernel(ei_hbm, out_hbm, ib, ones, zbuf, acc, sem):
        c = lax.axis_index("c")
        s = lax.axis_index("s")
        w = c * 16 + s

        _zero_rows(zbuf, _ZR, _DW)
        one = jnp.ones((16,), jnp.float32)

        def orow(i, carry):
            for j in range(_DW // 16):
                ones[i, pl.ds(j * 16, 16)] = one
            return carry

        lax.fori_loop(0, _CH, orow, None)
        _zero_acc(zbuf, acc, s)

        ng = ng_even + jnp.where(w < rem_g, 1, 0)
        nc = pl.multiple_of(ng * 8, 8)
        has_extra = w == rem_g

        def ldgroup(j, carry):
            rbase = pl.multiple_of(8 * (split * j + w), 8)
            dbase = pl.multiple_of(8 * j, 8)
            pltpu.sync_copy(ei_hbm.at[1, pl.ds(rbase, 8), :],
                            ib.at[pl.ds(dbase, 8), :])
            return carry

        lax.fori_loop(0, ng, ldgroup, None)

        @pl.when(has_extra)
        def _():
            pltpu.sync_copy(ei_hbm.at[1, pl.ds(8 * _NG, _XROW), :],
                            ib.at[pl.ds(nc, _XROW), :])

        plsc.subcore_barrier()

        ntot = nc + jnp.where(has_extra, _XROW, 0)

        def fire(k, carry):
            pltpu.async_copy(ones, acc.at[ib.at[k]], sem, add=True)
            return carry

        lax.fori_loop(0, ntot, fire, None)

        def drain(k, carry):
            pltpu.make_async_copy(ones, acc.at[ib.at[0]], sem).wait()
            return carry

        lax.fori_loop(0, ntot, drain, None)

        plsc.subcore_barrier()
        _writeback(acc, out_hbm, c, s)

    return deg_kernel


def _make_agg(feature_split):
    split = 16 if feature_split else 32
    ng_even = _NG // split
    rem_g = _NG - ng_even * split

    scratch = (
        [pltpu.VMEM((2, _CH), jnp.int32)] * 4 +
        [pltpu.VMEM((_CH, _W), jnp.float32)] * 2 +
        [pltpu.VMEM((_ZR, _W), jnp.float32),
         pltpu.VMEM_SHARED((_N, _W), jnp.float32)] +
        [pltpu.SemaphoreType.DMA] * 8
    )

    @functools.partial(
        pl.kernel,
        mesh=_mesh(),
        out_type=jax.ShapeDtypeStruct((2 * _N, _W), jnp.float32),
        scratch_types=scratch,
    )
    def agg_kernel(ei_hbm, h_hbm, out_hbm, ib0, ib1, ib2, ib3, r0, r1,
                   zbuf, acc, si0, si1, si2, si3, sg0, sg1, ss0, ss1):
        c = lax.axis_index("c")
        s = lax.axis_index("s")
        w = s if feature_split else c * 16 + s
        ibq = [ib0, ib1, ib2, ib3]
        sis = [si0, si1, si2, si3]
        rs = [r0, r1]
        sgs = [sg0, sg1]
        sss = [ss0, ss1]
        coff = c * _N

        _zero_rows(zbuf, _ZR, _W)
        _zero_acc(zbuf, acc, s)

        ng = ng_even + jnp.where(w < rem_g, 1, 0)
        nc = ng * 8
        has_extra = w == rem_g

        def ei_off(k):
            j = k // 8
            return pl.multiple_of((split * j + w) * 1024 + (k - 8 * j) * 128,
                                  128)

        def i_start(k, q):
            pltpu.async_copy(ei_hbm.at[:, pl.ds(ei_off(k), _CH)],
                             ibq[q], sis[q])

        def i_wait(k, q):
            pltpu.make_async_copy(ei_hbm.at[:, pl.ds(ei_off(k), _CH)],
                                  ibq[q], sis[q]).wait()

        def adj(q):
            if feature_split:
                for m in range(_CH // 16):
                    ibq[q][0, pl.ds(m * 16, 16)] = (
                        ibq[q][0, pl.ds(m * 16, 16)] + coff)

        def g_start(p, q):
            pltpu.async_copy(h_hbm.at[ibq[q].at[0]], rs[p], sgs[p])

        def g_wait(p, q):
            pltpu.make_async_copy(h_hbm.at[ibq[q].at[0]], rs[p], sgs[p]).wait()

        def s_start(p, q):
            pltpu.async_copy(rs[p], acc.at[ibq[q].at[1]], sss[p], add=True)

        def s_wait(p, q):
            pltpu.make_async_copy(rs[p], acc.at[ibq[q].at[1]], sss[p]).wait()

        plsc.subcore_barrier()

        i_start(0, 0)
        i_start(1, 1)
        i_start(2, 2)
        i_wait(0, 0)
        adj(0)
        g_start(0, 0)
        i_start(3, 3)
        i_wait(1, 1)
        adj(1)
        g_start(1, 1)
        g_wait(0, 0)
        s_start(0, 0)
        s_wait(0, 2)
        i_start(4, 0)
        i_wait(2, 2)
        adj(2)
        g_start(0, 2)
        g_wait(1, 1)
        s_start(1, 1)
        s_wait(1, 3)
        i_start(5, 1)
        i_wait(3, 3)
        adj(3)
        g_start(1, 3)
        g_wait(0, 2)
        s_start(0, 2)

        def quad(k4, carry):
            for q in range(4):
                k = k4 * 4 + q
                p = q % 2
                s_wait(p, q)

                @pl.when(k + 2 < nc)
                def _():
                    i_start(k + 2, (q + 2) % 4)

                i_wait(k, q)
                adj(q)
                g_start(p, q)
                g_wait(1 - p, (q + 3) % 4)
                s_start(1 - p, (q + 3) % 4)
            return carry

        lax.fori_loop(1, nc // 4, quad, None)

        g_wait(1, 3)
        s_start(1, 3)
        s_wait(0, 2)
        s_wait(1, 3)

        @pl.when(has_extra)
        def _():
            for j in range(_XROW):
                off = (8 * _NG + j) * 128
                pltpu.sync_copy(ei_hbm.at[:, pl.ds(off, _CH)], ibq[j])
                adj(j)
                pltpu.async_copy(h_hbm.at[ibq[j].at[0]], rs[j], sgs[j]).wait()
                pltpu.async_copy(rs[j], acc.at[ibq[j].at[1]], sss[j],
                                 add=True).wait()

        plsc.subcore_barrier()
        _writeback(acc, out_hbm, c, s)

    return agg_kernel


def _dinv_from(degp_ref):
    deg = degp_ref[0][:, 0] + degp_ref[1][:, 0] + 1.0
    return lax.rsqrt(deg)[:, None]


def _tc_a_body(x_ref, w_ref, degp_ref, out_ref):
    dinv = _dinv_from(degp_ref)
    h = jnp.dot(x_ref[...], w_ref[...], preferred_element_type=jnp.float32)
    hs = h * dinv
    out_ref[0] = hs[:, : _DH // 2]
    out_ref[1] = hs[:, _DH // 2:]


def _tc_b_body(agg_ref, hs_ref, degp_ref, b1_ref, w2_ref, out_ref):
    dinv = _dinv_from(degp_ref)
    a = jnp.concatenate([agg_ref[0] + hs_ref[0], agg_ref[1] + hs_ref[1]], axis=1)
    z = jnp.maximum(a * dinv + b1_ref[...], 0.0)
    h2 = jnp.dot(z, w2_ref[...], preferred_element_type=jnp.float32) * dinv
    out_ref[...] = jnp.concatenate(
        [h2, jnp.zeros((h2.shape[0], _W - _DOUT), jnp.float32)], axis=1)


def _tc_c_body(agg_ref, hs_ref, degp_ref, b2_ref, out_ref):
    dinv = _dinv_from(degp_ref)
    z = (agg_ref[0][:, : _DOUT] + agg_ref[1][:, : _DOUT]
         + hs_ref[:, : _DOUT]) * dinv + b2_ref[...]
    m = jnp.max(z, axis=1, keepdims=True)
    lse = jnp.log(jnp.sum(jnp.exp(z - m), axis=1, keepdims=True)) + m
    out_ref[...] = z - lse


def _node_spec(width):
    return pl.BlockSpec((_BN, width), lambda n: (n, 0))


def _split_spec(width):
    return pl.BlockSpec((2, _BN, width), lambda n: (0, n, 0))


def _full_spec(r, c):
    return pl.BlockSpec((r, c), lambda n: (0, 0))


def _tc_a(x, w1, degp):
    return pl.pallas_call(
        _tc_a_body,
        grid=(_N // _BN,),
        in_specs=[_node_spec(_DIN), _full_spec(_DIN, _DH), _split_spec(_DW)],
        out_specs=_split_spec(_DH // 2),
        out_shape=jax.ShapeDtypeStruct((2, _N, _DH // 2), jnp.float32),
    )(x, w1, degp)


def _tc_b(agg1, hs1, degp, b1, w2):
    return pl.pallas_call(
        _tc_b_body,
        grid=(_N // _BN,),
        in_specs=[_split_spec(_DH // 2), _split_spec(_DH // 2),
                  _split_spec(_DW), _full_spec(1, _DH), _full_spec(_DH, _DOUT)],
        out_specs=_node_spec(_W),
        out_shape=jax.ShapeDtypeStruct((_N, _W), jnp.float32),
    )(agg1, hs1, degp, b1, w2)


def _tc_c(agg2, hs2, degp, b2):
    return pl.pallas_call(
        _tc_c_body,
        grid=(_N // _BN,),
        in_specs=[_split_spec(_W), _node_spec(_W),
                  _split_spec(_DW), _full_spec(1, _DOUT)],
        out_specs=_node_spec(_DOUT),
        out_shape=jax.ShapeDtypeStruct((_N, _DOUT), jnp.float32),
    )(agg2, hs2, degp, b2)


def kernel(x, edge_index, W1, b1, W2, b2):
    ei3 = edge_index.reshape(2, _NR, _CH)

    degp = _make_deg()(ei3).reshape(2, _N, _DW)

    hs1 = _tc_a(x, W1, degp)
    agg1 = _make_agg(True)(edge_index, hs1.reshape(2 * _N, _DH // 2))
    agg1 = agg1.reshape(2, _N, _DH // 2)

    hs2 = _tc_b(agg1, hs1, degp, b1.reshape(1, _DH), W2)
    agg2 = _make_agg(False)(edge_index, hs2).reshape(2, _N, _W)

    return _tc_c(agg2, hs2, degp, b2.reshape(1, _DOUT))
